# baseline (speedup 1.0000x reference)
import numpy as np
import jax
import jax.numpy as jnp
from jax import lax
from jax.experimental import pallas as pl
from jax.experimental.pallas import tpu as pltpu

B, K, V = 64, 10, 100000
NROWS = B * K

_K0 = np.uint32(0)
_K1 = np.uint32(42)
_KS = (_K0, _K1, np.uint32(_K0 ^ _K1 ^ np.uint32(0x1BD11BDA)))
_ROT = (13, 15, 26, 6, 17, 29, 16, 24, 13, 15, 26, 6, 17, 29, 16, 24, 13, 15, 26, 6)
_TINY = np.float32(np.finfo(np.float32).tiny)
_NEG_INF = np.float32(-np.inf)

ROWS_PER_BLOCK = 8
GRID_R = NROWS // ROWS_PER_BLOCK
CHUNK = 512
UNROLL = 2
SBC = 25600
SB_WIDTHS = (SBC, SBC, SBC, 22528)
NSB = len(SB_WIDTHS)
TAIL_START = sum(SB_WIDTHS)
TAIL_LEN = V - TAIL_START
TAIL_BUF = 1024


def _i32(x):
    return np.int32(np.uint32(x))


def _sample_kernel(w_hbm, wtail_ref, out_ref, buf0, buf1, sem0, sem1):
    pid = pl.program_id(0)
    bufs = (buf0, buf1)
    sems = (sem0, sem1)
    shape = (ROWS_PER_BLOCK, CHUNK)
    row = lax.broadcasted_iota(jnp.int32, shape, 0) + pid * ROWS_PER_BLOCK
    lane = lax.broadcasted_iota(jnp.int32, shape, 1)
    tagb = row * V + lane + _i32(_KS[1])

    def copy(j):
        width = SB_WIDTHS[j]
        return pltpu.make_async_copy(
            w_hbm.at[pl.ds(pid * ROWS_PER_BLOCK, ROWS_PER_BLOCK),
                     pl.ds(sum(SB_WIDTHS[:j]), width)],
            bufs[j % 2].at[:, pl.ds(0, width)],
            sems[j % 2],
        )

    def rotl(x, r):
        return lax.shift_left(x, _i32(r)) | lax.shift_right_logical(x, _i32(32 - r))

    def chunk_score(w_ref, base, c, limit):
        w = w_ref[:, pl.ds(pl.multiple_of(c * CHUNK, 128), CHUNK)]
        tag = tagb + (base + c * CHUNK)
        x1 = tag
        x0 = x1
        x1 = rotl(x1, _ROT[0])
        x1 = x0 ^ x1
        for j in range(1, 4):
            x0 = x0 + x1
            x1 = rotl(x1, _ROT[j])
            x1 = x0 ^ x1
        x0 = x0 + _i32(_KS[1])
        x1 = x1 + _i32(np.uint32(_KS[2]) + np.uint32(1))
        for g in range(1, 5):
            for j in range(4):
                x0 = x0 + x1
                x1 = rotl(x1, _ROT[g * 4 + j])
                x1 = x0 ^ x1
            x0 = x0 + _i32(_KS[(g + 1) % 3])
            x1 = x1 + _i32(np.uint32(_KS[(g + 2) % 3]) + np.uint32(g + 1))
        bits = x0 ^ x1

        fb = lax.shift_right_logical(bits, _i32(9)) | _i32(0x3F800000)
        u = lax.bitcast_convert_type(fb, jnp.float32) - np.float32(1.0)
        u = jnp.maximum(u, _TINY)
        g = -jnp.log(-jnp.log(u))
        s = jnp.log(w + np.float32(1e-20)) + g
        if limit is not None:
            s = jnp.where(lane + c * CHUNK < limit, s, _NEG_INF)
        return s, tag

    def update(carry, s, tag):
        vmax, vtag = carry
        upd = s > vmax
        vmax = jnp.maximum(vmax, s)
        vtag = jnp.where(upd, tag, vtag)
        return vmax, vtag

    copy(0).start()
    carry = (jnp.full(shape, _NEG_INF, jnp.float32), jnp.zeros(shape, jnp.int32))
    for j in range(NSB):
        copy(j).wait()
        if j + 1 < NSB:
            copy(j + 1).start()
        w_ref = bufs[j % 2]
        base = sum(SB_WIDTHS[:j])
        nchunk = SB_WIDTHS[j] // CHUNK

        def body(c, carry, w_ref=w_ref, base=base):
            for k in range(UNROLL):
                carry = update(carry, *chunk_score(w_ref, base, c * UNROLL + k, None))
            return carry

        carry = lax.fori_loop(0, nchunk // UNROLL, body, carry)
        for c in range(nchunk - nchunk % UNROLL, nchunk):
            carry = update(carry, *chunk_score(w_ref, base, c, None))

    for c in range(TAIL_BUF // CHUNK):
        carry = update(carry, *chunk_score(wtail_ref, TAIL_START, c,
                                           TAIL_LEN))
    vmax, vtag = carry
    m = jnp.max(vmax, axis=1, keepdims=True)
    sel = jnp.where(vmax == m, vtag, np.int32(np.iinfo(np.int32).max))
    best_tag = jnp.min(sel, axis=1, keepdims=True)
    rowv = (lax.broadcasted_iota(jnp.int32, (ROWS_PER_BLOCK, 1), 0)
            + pid * ROWS_PER_BLOCK) * V + _i32(_KS[1])
    out_ref[0, 0, :] = (best_tag - rowv)[:, 0]


@jax.jit
def kernel(batch_k_head_softmax):
    w = batch_k_head_softmax.reshape(NROWS, V)
    wtail = jnp.pad(w[:, TAIL_START:], ((0, 0), (0, TAIL_BUF - TAIL_LEN)))
    out = pl.pallas_call(
        _sample_kernel,
        grid=(GRID_R,),
        in_specs=[
            pl.BlockSpec(memory_space=pl.ANY),
            pl.BlockSpec((ROWS_PER_BLOCK, TAIL_BUF), lambda i: (i, 0)),
        ],
        out_specs=pl.BlockSpec((1, 1, ROWS_PER_BLOCK), lambda i: (i, 0, 0)),
        out_shape=jax.ShapeDtypeStruct((GRID_R, 1, ROWS_PER_BLOCK), jnp.int32),
        scratch_shapes=[
            pltpu.VMEM((ROWS_PER_BLOCK, SBC), jnp.float32),
            pltpu.VMEM((ROWS_PER_BLOCK, SBC), jnp.float32),
            pltpu.SemaphoreType.DMA,
            pltpu.SemaphoreType.DMA,
        ],
        compiler_params=pltpu.CompilerParams(
            dimension_semantics=("arbitrary",),
        ),
    )(w, wtail)
    return out.reshape(B, K)


# speedup vs baseline: 1.1313x; 1.1313x over previous
import numpy as np
import jax
import jax.numpy as jnp
from jax import lax
from jax.experimental import pallas as pl
from jax.experimental.pallas import tpu as pltpu

B, K, V = 64, 10, 100000
NROWS = B * K

_K0 = np.uint32(0)
_K1 = np.uint32(42)
_KS = (_K0, _K1, np.uint32(_K0 ^ _K1 ^ np.uint32(0x1BD11BDA)))
_ROT = (13, 15, 26, 6, 17, 29, 16, 24, 13, 15, 26, 6, 17, 29, 16, 24, 13, 15, 26, 6)
_TINY = np.float32(np.finfo(np.float32).tiny)
_NEG_INF = np.float32(-np.inf)

ROWS_PER_BLOCK = 8
GRID = NROWS // ROWS_PER_BLOCK
CHUNK = 1024
UNROLL = 2
NFULL = V // CHUNK
TAIL_START = NFULL * CHUNK
TAIL_LEN = V - TAIL_START


def _i32(x):
    return np.int32(np.uint32(x))


def _sample_kernel(w_hbm, wtail_ref, out_ref, buf0, buf1, sem0, sem1):
    pid = pl.program_id(0)
    bufs = (buf0, buf1)
    sems = (sem0, sem1)
    shape = (ROWS_PER_BLOCK, CHUNK)
    row = lax.broadcasted_iota(jnp.int32, shape, 0) + pid * ROWS_PER_BLOCK
    tag0 = row * V + lax.broadcasted_iota(jnp.int32, shape, 1) + _i32(_KS[1])

    def copy(block, slot):
        return pltpu.make_async_copy(
            w_hbm.at[pl.ds(block * ROWS_PER_BLOCK, ROWS_PER_BLOCK), :],
            bufs[slot],
            sems[slot],
        )

    @pl.when(pid == 0)
    def _():
        copy(pid, 0).start()

    parity = lax.rem(pid, 2)

    @pl.when(parity == 0)
    def _():
        copy(pid, 0).wait()

    @pl.when(parity == 1)
    def _():
        copy(pid, 1).wait()

    @pl.when(jnp.logical_and(pid + 1 < GRID, parity == 0))
    def _():
        copy(pid + 1, 1).start()

    @pl.when(jnp.logical_and(pid + 1 < GRID, parity == 1))
    def _():
        copy(pid + 1, 0).start()

    def rotl(x, r):
        return lax.shift_left(x, _i32(r)) | lax.shift_right_logical(x, _i32(32 - r))

    def chunk_score(start, w):
        tag = tag0 + start
        x1 = tag
        x0 = x1
        x1 = rotl(x1, _ROT[0])
        x1 = x0 ^ x1
        for j in range(1, 4):
            x0 = x0 + x1
            x1 = rotl(x1, _ROT[j])
            x1 = x0 ^ x1
        x0 = x0 + _i32(_KS[1])
        x1 = x1 + _i32(np.uint32(_KS[2]) + np.uint32(1))
        for g in range(1, 5):
            for j in range(4):
                x0 = x0 + x1
                x1 = rotl(x1, _ROT[g * 4 + j])
                x1 = x0 ^ x1
            x0 = x0 + _i32(_KS[(g + 1) % 3])
            x1 = x1 + _i32(np.uint32(_KS[(g + 2) % 3]) + np.uint32(g + 1))
        bits = x0 ^ x1

        fb = lax.shift_right_logical(bits, _i32(9)) | _i32(0x3F800000)
        u = lax.bitcast_convert_type(fb, jnp.float32) - np.float32(1.0)
        u = jnp.maximum(u, _TINY)
        g = -jnp.log(-jnp.log(u))
        s = jnp.log(w + np.float32(1e-20)) + g
        return s, tag

    def update(carry, s, tag):
        vmax, vtag = carry
        upd = s > vmax
        vmax = jnp.maximum(vmax, s)
        vtag = jnp.where(upd, tag, vtag)
        return vmax, vtag

    def run_block(w_ref, carry):
        def body(c, carry):
            for k in range(UNROLL):
                start = pl.multiple_of((c * UNROLL + k) * CHUNK, 128)
                carry = update(carry, *chunk_score(start, w_ref[:, pl.ds(start, CHUNK)]))
            return carry

        carry = lax.fori_loop(0, NFULL // UNROLL, body, carry)
        for c in range(NFULL - NFULL % UNROLL, NFULL):
            start = c * CHUNK
            carry = update(carry, *chunk_score(start, w_ref[:, pl.ds(start, CHUNK)]))
        return carry

    carry0 = (jnp.full(shape, _NEG_INF, jnp.float32), jnp.zeros(shape, jnp.int32))
    carry = lax.cond(parity == 0,
                     lambda c: run_block(buf0, c),
                     lambda c: run_block(buf1, c),
                     carry0)

    s, tag = chunk_score(TAIL_START, wtail_ref[...])
    lane = lax.broadcasted_iota(jnp.int32, shape, 1)
    s = jnp.where(lane < TAIL_LEN, s, _NEG_INF)
    vmax, vtag = update(carry, s, tag)

    m = jnp.max(vmax, axis=1, keepdims=True)
    sel = jnp.where(vmax == m, vtag, np.int32(np.iinfo(np.int32).max))
    best_tag = jnp.min(sel, axis=1, keepdims=True)
    rowv = (lax.broadcasted_iota(jnp.int32, (ROWS_PER_BLOCK, 1), 0)
            + pid * ROWS_PER_BLOCK) * V + _i32(_KS[1])
    out_ref[0, 0, :] = (best_tag - rowv)[:, 0]


@jax.jit
def kernel(batch_k_head_softmax):
    w = batch_k_head_softmax.reshape(NROWS, V)
    wtail = jnp.pad(w[:, TAIL_START:], ((0, 0), (0, CHUNK - TAIL_LEN)))
    out = pl.pallas_call(
        _sample_kernel,
        grid=(GRID,),
        in_specs=[
            pl.BlockSpec(memory_space=pl.ANY),
            pl.BlockSpec((ROWS_PER_BLOCK, CHUNK), lambda i: (i, 0)),
        ],
        out_specs=pl.BlockSpec((1, 1, ROWS_PER_BLOCK), lambda i: (i, 0, 0)),
        out_shape=jax.ShapeDtypeStruct((GRID, 1, ROWS_PER_BLOCK), jnp.int32),
        scratch_shapes=[
            pltpu.VMEM((ROWS_PER_BLOCK, V), jnp.float32),
            pltpu.VMEM((ROWS_PER_BLOCK, V), jnp.float32),
            pltpu.SemaphoreType.DMA,
            pltpu.SemaphoreType.DMA,
        ],
        compiler_params=pltpu.CompilerParams(
            dimension_semantics=("arbitrary",),
        ),
    )(w, wtail)
    return out.reshape(B, K)
